# baseline (speedup 1.0000x reference)
import functools

import jax
import jax.numpy as jnp
from jax import lax
from jax.experimental import pallas as pl
from jax.experimental.pallas import tpu as pltpu
from jax.experimental.pallas import tpu_sc as plsc

DIM = 128
BATCH = 16384

_info = plsc.get_sparse_core_info()
_NC, _NS = _info.num_cores, _info.num_subcores
_NW = _NC * _NS
_B_PER_W = BATCH // _NW

_mesh = plsc.VectorSubcoreMesh(core_axis_name="c", subcore_axis_name="s")

_NCHUNK = 4
_C = _B_PER_W // _NCHUNK


@functools.partial(
    pl.kernel,
    mesh=_mesh,
    out_type=jax.ShapeDtypeStruct((BATCH, DIM), jnp.float32),
    scratch_types=[
        pltpu.VMEM((_B_PER_W,), jnp.int32),
        pltpu.VMEM((_B_PER_W, DIM), jnp.float32),
    ]
    + [pltpu.SemaphoreType.DMA] * (2 * _NCHUNK),
)
def _gather_kernel(idx_hbm, table_hbm, out_hbm, idx_v, rows_v, *sems):
    gsems, ssems = sems[:_NCHUNK], sems[_NCHUNK:]
    wid = lax.axis_index("s") * _NC + lax.axis_index("c")
    base = wid * _B_PER_W
    pltpu.sync_copy(idx_hbm.at[pl.ds(base, _B_PER_W)], idx_v)
    gathers = [
        pltpu.async_copy(
            table_hbm.at[idx_v.at[pl.ds(i * _C, _C)]],
            rows_v.at[pl.ds(i * _C, _C)],
            gsems[i],
        )
        for i in range(_NCHUNK)
    ]
    stores = []
    for i in range(_NCHUNK):
        gathers[i].wait()
        stores.append(
            pltpu.async_copy(
                rows_v.at[pl.ds(i * _C, _C)],
                out_hbm.at[pl.ds(base + i * _C, _C)],
                ssems[i],
            )
        )
    for s in stores:
        s.wait()


def kernel(x, embedding):
    return _gather_kernel(x.astype(jnp.int32), embedding)
